# Initial kernel scaffold; baseline (speedup 1.0000x reference)
import jax
import jax.numpy as jnp
from jax.experimental import pallas as pl


def kernel(x, W_hash, basis, W_dec):
    raise NotImplementedError("write your pallas kernel here")


# speedup vs baseline: 3.0466x; 3.0466x over previous
import jax
import jax.numpy as jnp
from jax.experimental import pallas as pl

_D_MODEL = 768
_D_EMB = 64
_E = 64
_BASIS = 8
_EB = _E * _BASIS
_TM = 512


def _hrn_block(x_ref, wh_ref, basis_ref, wdec_ref, y_ref):
    f32 = jnp.float32
    hp = jax.lax.Precision.HIGHEST

    x = x_ref[...]
    wh = wh_ref[...]
    basis = basis_ref[...]
    wdec = wdec_ref[...]

    tm = x.shape[0]

    e = jax.lax.dot_general(x, wh, (((1,), (0,)), ((), ())),
                            preferred_element_type=f32)
    e = e / (jnp.sqrt(jnp.sum(e * e, axis=1, keepdims=True)) + 1e-8)

    flat = basis / (jnp.sqrt(jnp.sum(basis * basis, axis=1, keepdims=True)) + 1e-8)

    coeffs = jax.lax.dot_general(e, flat, (((1,), (1,)), ((), ())),
                                 preferred_element_type=f32)

    row = jax.lax.broadcasted_iota(jnp.int32, (_EB, _E), 0) // _BASIS
    col = jax.lax.broadcasted_iota(jnp.int32, (_EB, _E), 1)
    seg = (row == col).astype(f32)
    energy = jax.lax.dot_general(coeffs * coeffs, seg, (((1,), (0,)), ((), ())),
                                 precision=hp, preferred_element_type=f32)

    idx = jax.lax.broadcasted_iota(jnp.int32, (tm, _E), 1)
    m1 = jnp.max(energy, axis=1, keepdims=True)
    a1 = jnp.min(jnp.where(energy == m1, idx, _E), axis=1, keepdims=True)
    en2 = jnp.where(idx == a1, -1.0, energy)
    m2 = jnp.max(en2, axis=1, keepdims=True)
    a2 = jnp.min(jnp.where(en2 == m2, idx, _E), axis=1, keepdims=True)
    ed = jnp.exp(m2 - m1)
    g1 = 1.0 / (1.0 + ed)
    g2 = ed / (1.0 + ed)
    w = jnp.where(idx == a1, g1, 0.0) + jnp.where(idx == a2, g2, 0.0)

    w8 = jax.lax.dot_general(w, seg, (((1,), (1,)), ((), ())),
                             precision=hp, preferred_element_type=f32)
    mix = jax.lax.dot_general(coeffs * w8, flat, (((1,), (0,)), ((), ())),
                              precision=hp, preferred_element_type=f32)

    y_ref[...] = jax.lax.dot_general(mix, wdec, (((1,), (0,)), ((), ())),
                                     precision=hp, preferred_element_type=f32)


@jax.jit
def kernel(x, W_hash, basis, W_dec):
    t = x.shape[0]
    basis2 = basis.reshape(_EB, _D_EMB)
    return pl.pallas_call(
        _hrn_block,
        grid=(t // _TM,),
        in_specs=[
            pl.BlockSpec((_TM, _D_MODEL), lambda i: (i, 0)),
            pl.BlockSpec((_D_MODEL, _D_EMB), lambda i: (0, 0)),
            pl.BlockSpec((_EB, _D_EMB), lambda i: (0, 0)),
            pl.BlockSpec((_D_EMB, _D_MODEL), lambda i: (0, 0)),
        ],
        out_specs=pl.BlockSpec((_TM, _D_MODEL), lambda i: (i, 0)),
        out_shape=jax.ShapeDtypeStruct((t, _D_MODEL), jnp.float32),
    )(x, W_hash, basis2, W_dec)


# speedup vs baseline: 5.0737x; 1.6653x over previous
import jax
import jax.numpy as jnp
from jax.experimental import pallas as pl

_D_MODEL = 768
_D_EMB = 64
_E = 64
_BASIS = 8
_EB = _E * _BASIS
_TM = 512


def _hrn_block(x_ref, wh_ref, basis_ref, wdec_ref, y_ref):
    f32 = jnp.float32

    x = x_ref[...]
    wh = wh_ref[...]
    basis = basis_ref[...]
    wdec = wdec_ref[...]

    tm = x.shape[0]

    e = jax.lax.dot_general(x, wh, (((1,), (0,)), ((), ())),
                            preferred_element_type=f32)
    e = e / (jnp.sqrt(jnp.sum(e * e, axis=1, keepdims=True)) + 1e-8)

    flat = basis / (jnp.sqrt(jnp.sum(basis * basis, axis=1, keepdims=True)) + 1e-8)

    coeffs = jax.lax.dot_general(e, flat, (((1,), (1,)), ((), ())),
                                 preferred_element_type=f32)

    sq = coeffs * coeffs
    energy = sq[:, 0:_E]
    for b in range(1, _BASIS):
        energy = energy + sq[:, b * _E:(b + 1) * _E]

    idx = jax.lax.broadcasted_iota(jnp.int32, (tm, _E), 1)
    m1 = jnp.max(energy, axis=1, keepdims=True)
    a1 = jnp.min(jnp.where(energy == m1, idx, _E), axis=1, keepdims=True)
    en2 = jnp.where(idx == a1, -1.0, energy)
    m2 = jnp.max(en2, axis=1, keepdims=True)
    a2 = jnp.min(jnp.where(en2 == m2, idx, _E), axis=1, keepdims=True)
    ed = jnp.exp(m2 - m1)
    g1 = 1.0 / (1.0 + ed)
    g2 = ed / (1.0 + ed)
    w = jnp.where(idx == a1, g1, 0.0) + jnp.where(idx == a2, g2, 0.0)

    w8 = jnp.concatenate([w] * _BASIS, axis=1)
    mix = jax.lax.dot_general(coeffs * w8, flat, (((1,), (0,)), ((), ())),
                              precision=jax.lax.Precision.HIGHEST,
                              preferred_element_type=f32)

    y_ref[...] = jax.lax.dot_general(mix, wdec, (((1,), (0,)), ((), ())),
                                     preferred_element_type=f32)


@jax.jit
def kernel(x, W_hash, basis, W_dec):
    t = x.shape[0]
    basis2 = basis.transpose(1, 0, 2).reshape(_EB, _D_EMB)
    return pl.pallas_call(
        _hrn_block,
        grid=(t // _TM,),
        in_specs=[
            pl.BlockSpec((_TM, _D_MODEL), lambda i: (i, 0)),
            pl.BlockSpec((_D_MODEL, _D_EMB), lambda i: (0, 0)),
            pl.BlockSpec((_EB, _D_EMB), lambda i: (0, 0)),
            pl.BlockSpec((_D_EMB, _D_MODEL), lambda i: (0, 0)),
        ],
        out_specs=pl.BlockSpec((_TM, _D_MODEL), lambda i: (i, 0)),
        out_shape=jax.ShapeDtypeStruct((t, _D_MODEL), jnp.float32),
    )(x, W_hash, basis2, W_dec)


# speedup vs baseline: 6.3906x; 1.2596x over previous
import jax
import jax.numpy as jnp
from jax.experimental import pallas as pl

_D_MODEL = 768
_D_EMB = 64
_E = 64
_BASIS = 8
_EB = _E * _BASIS
_TM = 1024


def _hrn_block(x_ref, wh_ref, basis_ref, wdec_ref, y_ref):
    f32 = jnp.float32

    x = x_ref[...]
    wh = wh_ref[...]
    basis = basis_ref[...]
    wdec = wdec_ref[...]

    tm = x.shape[0]

    e = jax.lax.dot_general(x, wh, (((1,), (0,)), ((), ())),
                            preferred_element_type=f32)
    e = e / (jnp.sqrt(jnp.sum(e * e, axis=1, keepdims=True)) + 1e-8)

    flat = basis / (jnp.sqrt(jnp.sum(basis * basis, axis=1, keepdims=True)) + 1e-8)

    coeffs = jax.lax.dot_general(e, flat, (((1,), (1,)), ((), ())),
                                 preferred_element_type=f32)

    sq = coeffs * coeffs
    energy = sq[:, 0:_E]
    for b in range(1, _BASIS):
        energy = energy + sq[:, b * _E:(b + 1) * _E]

    idx = jax.lax.broadcasted_iota(jnp.int32, (tm, _E), 1).astype(f32)
    m1 = jnp.max(energy, axis=1, keepdims=True)
    a1 = jnp.min(jnp.where(energy == m1, idx, 64.0), axis=1, keepdims=True)
    en2 = jnp.where(idx == a1, -1.0, energy)
    m2 = jnp.max(en2, axis=1, keepdims=True)
    a2 = jnp.min(jnp.where(en2 == m2, idx, 64.0), axis=1, keepdims=True)
    ed = jnp.exp(m2 - m1)
    g1 = 1.0 / (1.0 + ed)
    g2 = ed / (1.0 + ed)
    w = jnp.where(idx == a1, g1, 0.0) + jnp.where(idx == a2, g2, 0.0)

    w8 = jnp.concatenate([w] * _BASIS, axis=1)
    mix = jax.lax.dot_general(coeffs * w8, flat, (((1,), (0,)), ((), ())),
                              preferred_element_type=f32)

    y_ref[...] = jax.lax.dot_general(mix, wdec, (((1,), (0,)), ((), ())),
                                     preferred_element_type=f32)


@jax.jit
def kernel(x, W_hash, basis, W_dec):
    t = x.shape[0]
    basis2 = basis.transpose(1, 0, 2).reshape(_EB, _D_EMB)
    return pl.pallas_call(
        _hrn_block,
        grid=(t // _TM,),
        in_specs=[
            pl.BlockSpec((_TM, _D_MODEL), lambda i: (i, 0)),
            pl.BlockSpec((_D_MODEL, _D_EMB), lambda i: (0, 0)),
            pl.BlockSpec((_EB, _D_EMB), lambda i: (0, 0)),
            pl.BlockSpec((_D_EMB, _D_MODEL), lambda i: (0, 0)),
        ],
        out_specs=pl.BlockSpec((_TM, _D_MODEL), lambda i: (i, 0)),
        out_shape=jax.ShapeDtypeStruct((t, _D_MODEL), jnp.float32),
    )(x, W_hash, basis2, W_dec)


# speedup vs baseline: 9.6098x; 1.5037x over previous
import jax
import jax.numpy as jnp
from jax.experimental import pallas as pl

_D_MODEL = 768
_D_EMB = 64
_E = 64
_BASIS = 8
_EB = _E * _BASIS
_TM = 1024


def _hrn_block(x_ref, wh_ref, basis_ref, wdec_ref, y_ref):
    f32 = jnp.float32

    x = x_ref[...]
    wh = wh_ref[...]
    basis = basis_ref[...]
    wdec = wdec_ref[...]

    tm = x.shape[0]

    e = jax.lax.dot_general(x, wh, (((1,), (0,)), ((), ())),
                            preferred_element_type=f32)
    e = e / (jnp.sqrt(jnp.sum(e * e, axis=1, keepdims=True)) + 1e-8)

    flat = basis / (jnp.sqrt(jnp.sum(basis * basis, axis=1, keepdims=True)) + 1e-8)

    coeffs = jax.lax.dot_general(e, flat, (((1,), (1,)), ((), ())),
                                 preferred_element_type=f32)

    sq = coeffs * coeffs
    s4 = sq[:, 0:4 * _E] + sq[:, 4 * _E:8 * _E]
    s2 = s4[:, 0:2 * _E] + s4[:, 2 * _E:4 * _E]
    energy = s2[:, 0:_E] + s2[:, _E:2 * _E]

    m1 = jnp.max(energy, axis=1, keepdims=True)
    is1 = energy == m1
    en2 = jnp.where(is1, -1.0, energy)
    m2 = jnp.max(en2, axis=1, keepdims=True)
    ed = jnp.exp(m2 - m1)
    g1 = 1.0 / (1.0 + ed)
    g2 = ed / (1.0 + ed)
    w = jnp.where(is1, g1, jnp.where(en2 == m2, g2, 0.0))

    w8 = jnp.concatenate([w] * _BASIS, axis=1)
    mix = jax.lax.dot_general(coeffs * w8, flat, (((1,), (0,)), ((), ())),
                              preferred_element_type=f32)

    y_ref[...] = jax.lax.dot_general(mix, wdec, (((1,), (0,)), ((), ())),
                                     preferred_element_type=f32)


@jax.jit
def kernel(x, W_hash, basis, W_dec):
    t = x.shape[0]
    basis2 = basis.transpose(1, 0, 2).reshape(_EB, _D_EMB)
    return pl.pallas_call(
        _hrn_block,
        grid=(t // _TM,),
        in_specs=[
            pl.BlockSpec((_TM, _D_MODEL), lambda i: (i, 0)),
            pl.BlockSpec((_D_MODEL, _D_EMB), lambda i: (0, 0)),
            pl.BlockSpec((_EB, _D_EMB), lambda i: (0, 0)),
            pl.BlockSpec((_D_EMB, _D_MODEL), lambda i: (0, 0)),
        ],
        out_specs=pl.BlockSpec((_TM, _D_MODEL), lambda i: (i, 0)),
        out_shape=jax.ShapeDtypeStruct((t, _D_MODEL), jnp.float32),
    )(x, W_hash, basis2, W_dec)


# speedup vs baseline: 9.8751x; 1.0276x over previous
import jax
import jax.numpy as jnp
from jax.experimental import pallas as pl
from jax.experimental.pallas import tpu as pltpu

_D_MODEL = 768
_D_EMB = 64
_E = 64
_BASIS = 8
_EB = _E * _BASIS
_TM = 1024
_HALVES = 2


def _half(x, wh, flat, wdec, y_ref, r0, rows):
    f32 = jnp.float32

    e = jax.lax.dot_general(x, wh, (((1,), (0,)), ((), ())),
                            preferred_element_type=f32)
    e = e / (jnp.sqrt(jnp.sum(e * e, axis=1, keepdims=True)) + 1e-8)

    coeffs = jax.lax.dot_general(e, flat, (((1,), (1,)), ((), ())),
                                 preferred_element_type=f32)

    sq = coeffs * coeffs
    s4 = sq[:, 0:4 * _E] + sq[:, 4 * _E:8 * _E]
    s2 = s4[:, 0:2 * _E] + s4[:, 2 * _E:4 * _E]
    energy = s2[:, 0:_E] + s2[:, _E:2 * _E]

    m1 = jnp.max(energy, axis=1, keepdims=True)
    is1 = energy == m1
    en2 = jnp.where(is1, -1.0, energy)
    m2 = jnp.max(en2, axis=1, keepdims=True)
    ed = jnp.exp(m2 - m1)
    g1 = 1.0 / (1.0 + ed)
    g2 = ed / (1.0 + ed)
    w = jnp.where(is1, g1, jnp.where(en2 == m2, g2, 0.0))

    w8 = jnp.concatenate([w] * _BASIS, axis=1)
    mix = jax.lax.dot_general(coeffs * w8, flat, (((1,), (0,)), ((), ())),
                              preferred_element_type=f32)

    y_ref[pl.ds(r0, rows), :] = jax.lax.dot_general(
        mix, wdec, (((1,), (0,)), ((), ())), preferred_element_type=f32)


def _hrn_block(x_ref, wh_ref, basis_ref, wdec_ref, y_ref, flat_ref):
    @pl.when(pl.program_id(0) == 0)
    def _init():
        basis = basis_ref[...]
        flat_ref[...] = basis / (
            jnp.sqrt(jnp.sum(basis * basis, axis=1, keepdims=True)) + 1e-8)

    wh = wh_ref[...]
    wdec = wdec_ref[...]
    flat = flat_ref[...]

    rows = _TM // _HALVES
    for h in range(_HALVES):
        _half(x_ref[pl.ds(h * rows, rows), :], wh, flat, wdec,
              y_ref, h * rows, rows)


@jax.jit
def kernel(x, W_hash, basis, W_dec):
    t = x.shape[0]
    basis2 = basis.transpose(1, 0, 2).reshape(_EB, _D_EMB)
    return pl.pallas_call(
        _hrn_block,
        grid=(t // _TM,),
        in_specs=[
            pl.BlockSpec((_TM, _D_MODEL), lambda i: (i, 0)),
            pl.BlockSpec((_D_MODEL, _D_EMB), lambda i: (0, 0)),
            pl.BlockSpec((_EB, _D_EMB), lambda i: (0, 0)),
            pl.BlockSpec((_D_EMB, _D_MODEL), lambda i: (0, 0)),
        ],
        out_specs=pl.BlockSpec((_TM, _D_MODEL), lambda i: (i, 0)),
        out_shape=jax.ShapeDtypeStruct((t, _D_MODEL), jnp.float32),
        scratch_shapes=[pltpu.VMEM((_EB, _D_EMB), jnp.float32)],
    )(x, W_hash, basis2, W_dec)


# speedup vs baseline: 9.9845x; 1.0111x over previous
import jax
import jax.numpy as jnp
from jax.experimental import pallas as pl
from jax.experimental.pallas import tpu as pltpu

_D_MODEL = 768
_D_EMB = 64
_E = 64
_BASIS = 8
_EB = _E * _BASIS
_TM = 2048
_HALVES = 4


def _half(x, wh, flat, wdec, y_ref, r0, rows):
    f32 = jnp.float32

    e = jax.lax.dot_general(x, wh, (((1,), (0,)), ((), ())),
                            preferred_element_type=f32)
    e = e * (1.0 / (jnp.sqrt(jnp.sum(e * e, axis=1, keepdims=True)) + 1e-8))

    coeffs = jax.lax.dot_general(e, flat, (((1,), (1,)), ((), ())),
                                 preferred_element_type=f32)

    sq = coeffs * coeffs
    s4 = sq[:, 0:4 * _E] + sq[:, 4 * _E:8 * _E]
    s2 = s4[:, 0:2 * _E] + s4[:, 2 * _E:4 * _E]
    energy = s2[:, 0:_E] + s2[:, _E:2 * _E]

    m1 = jnp.max(energy, axis=1, keepdims=True)
    is1 = energy == m1
    en2 = jnp.where(is1, -1.0, energy)
    m2 = jnp.max(en2, axis=1, keepdims=True)
    ed = jnp.exp(m2 - m1)
    g1 = 1.0 / (1.0 + ed)
    g2 = ed * g1
    w = jnp.where(is1, g1, jnp.where(en2 == m2, g2, 0.0))

    w8 = jnp.concatenate([w] * _BASIS, axis=1)
    mix = jax.lax.dot_general(coeffs * w8, flat, (((1,), (0,)), ((), ())),
                              preferred_element_type=f32)

    y_ref[pl.ds(r0, rows), :] = jax.lax.dot_general(
        mix, wdec, (((1,), (0,)), ((), ())), preferred_element_type=f32)


def _hrn_block(x_ref, wh_ref, basis_ref, wdec_ref, y_ref, flat_ref):
    @pl.when(pl.program_id(0) == 0)
    def _init():
        basis = basis_ref[...]
        flat_ref[...] = basis * (1.0 / (
            jnp.sqrt(jnp.sum(basis * basis, axis=1, keepdims=True)) + 1e-8))

    wh = wh_ref[...]
    wdec = wdec_ref[...]
    flat = flat_ref[...]

    rows = _TM // _HALVES
    for h in range(_HALVES):
        _half(x_ref[pl.ds(h * rows, rows), :], wh, flat, wdec,
              y_ref, h * rows, rows)


@jax.jit
def kernel(x, W_hash, basis, W_dec):
    t = x.shape[0]
    basis2 = basis.transpose(1, 0, 2).reshape(_EB, _D_EMB)
    return pl.pallas_call(
        _hrn_block,
        grid=(t // _TM,),
        in_specs=[
            pl.BlockSpec((_TM, _D_MODEL), lambda i: (i, 0)),
            pl.BlockSpec((_D_MODEL, _D_EMB), lambda i: (0, 0)),
            pl.BlockSpec((_EB, _D_EMB), lambda i: (0, 0)),
            pl.BlockSpec((_D_EMB, _D_MODEL), lambda i: (0, 0)),
        ],
        out_specs=pl.BlockSpec((_TM, _D_MODEL), lambda i: (i, 0)),
        out_shape=jax.ShapeDtypeStruct((t, _D_MODEL), jnp.float32),
        scratch_shapes=[pltpu.VMEM((_EB, _D_EMB), jnp.float32)],
    )(x, W_hash, basis2, W_dec)
